# Initial kernel scaffold; baseline (speedup 1.0000x reference)
import jax
import jax.numpy as jnp
from jax.experimental import pallas as pl


def kernel(x, edge_index, W1, b1, W2, b2):
    raise NotImplementedError("write your pallas kernel here")


# speedup vs baseline: 7.6202x; 7.6202x over previous
import functools

import jax
import jax.numpy as jnp
from jax import lax
from jax.experimental import pallas as pl
from jax.experimental.pallas import tpu as pltpu
from jax.experimental.pallas import tpu_sc as plsc

_N = 10000
_D = 256
_DH = 128
_E = 160000
_NC = 2
_NS = 16

_EPT = _E // _NS
_K = 80
_NCHUNK = _EPT // _K
_NP = 10240
_RPT = _NP // _NS
_RC = 128
_NRC = _RPT // _RC

_EPT32 = _E // (_NC * _NS)
_KD = 40
_NCD = _EPT32 // _KD
_DW = 128

_MESH = plsc.VectorSubcoreMesh(core_axis_name="c", subcore_axis_name="s")


def _deg_body(dst_hbm, ones_hbm, zeros_hbm, out_hbm,
              acc, didx_v, ones_v, tmp_v):
    c = lax.axis_index("c")
    s = lax.axis_index("s")
    pltpu.sync_copy(ones_hbm, ones_v)
    pltpu.sync_copy(zeros_hbm, tmp_v)
    for i in range(_NRC):
        pltpu.sync_copy(tmp_v, acc.at[pl.ds(s * _RPT + i * _RC, _RC)])
    plsc.subcore_barrier()
    wid = s * _NC + c
    ebase = wid * _EPT32

    def chunk(k, carry):
        start = pl.multiple_of(ebase + k * _KD, 8)
        pltpu.sync_copy(dst_hbm.at[pl.ds(start, _KD)], didx_v)
        pltpu.sync_copy(ones_v, acc.at[didx_v], add=True)
        return carry

    lax.fori_loop(0, _NCD, chunk, 0)
    plsc.subcore_barrier()
    for i in range(_NRC):
        r0 = s * _RPT + i * _RC
        pltpu.sync_copy(acc.at[pl.ds(r0, _RC)], tmp_v)
        pltpu.sync_copy(tmp_v, out_hbm.at[pl.ds(c * _NP + r0, _RC)])


_deg_call = pl.kernel(
    _deg_body,
    out_type=jax.ShapeDtypeStruct((_NC * _NP, _DW), jnp.float32),
    mesh=_MESH,
    scratch_types=[
        pltpu.VMEM_SHARED((_NP, _DW), jnp.float32),
        pltpu.VMEM((_KD,), jnp.int32),
        pltpu.VMEM((_KD, _DW), jnp.float32),
        pltpu.VMEM((_RC, _DW), jnp.float32),
    ],
)


def _agg_body(h_hbm, src2_hbm, dst_hbm, zeros_hbm, out_hbm,
              acc, sidx_v, didx_v, rows_v, tmp_v, sem):
    c = lax.axis_index("c")
    s = lax.axis_index("s")
    pltpu.sync_copy(zeros_hbm, tmp_v)
    for i in range(_NRC):
        pltpu.sync_copy(tmp_v, acc.at[pl.ds(s * _RPT + i * _RC, _RC)])
    plsc.subcore_barrier()
    ebase = c * _E + s * _EPT
    dbase = s * _EPT

    def chunk(k, carry):
        soff = pl.multiple_of(ebase + k * _K, 8)
        doff = pl.multiple_of(dbase + k * _K, 8)
        pltpu.sync_copy(src2_hbm.at[pl.ds(soff, _K)], sidx_v)
        pltpu.sync_copy(dst_hbm.at[pl.ds(doff, _K)], didx_v)
        pltpu.async_copy(h_hbm.at[sidx_v], rows_v, sem).wait()
        pltpu.sync_copy(rows_v, acc.at[didx_v], add=True)
        return carry

    lax.fori_loop(0, _NCHUNK, chunk, 0)
    plsc.subcore_barrier()
    for i in range(_NRC):
        r0 = s * _RPT + i * _RC
        pltpu.sync_copy(acc.at[pl.ds(r0, _RC)], tmp_v)
        pltpu.sync_copy(tmp_v, out_hbm.at[pl.ds(c * _NP + r0, _RC)])


_agg_call = pl.kernel(
    _agg_body,
    out_type=jax.ShapeDtypeStruct((_NC * _NP, _DH), jnp.float32),
    mesh=_MESH,
    scratch_types=[
        pltpu.VMEM_SHARED((_NP, _DH), jnp.float32),
        pltpu.VMEM((_K,), jnp.int32),
        pltpu.VMEM((_K,), jnp.int32),
        pltpu.VMEM((_K, _DH), jnp.float32),
        pltpu.VMEM((_RC, _DH), jnp.float32),
        pltpu.SemaphoreType.DMA,
    ],
)


_B = 1000
_G = _N // _B


def _dinv_from(dg_ref):
    dsum = dg_ref[0, :, 0:1] + dg_ref[1, :, 0:1] + 1.0
    return lax.rsqrt(dsum)


def _lin1_body(x_ref, w_ref, dg_ref, out_ref):
    h = jnp.dot(x_ref[...], w_ref[...], preferred_element_type=jnp.float32)
    ht = h * _dinv_from(dg_ref)
    out_ref[0] = ht[:, :_DH]
    out_ref[1] = ht[:, _DH:]


def _lin2_body(s_ref, h_ref, dg_ref, w_ref, b1_ref, out_ref):
    dinv = _dinv_from(dg_ref)
    xa = jnp.maximum(dinv * (s_ref[0] + h_ref[0]) + b1_ref[:, :_DH], 0.0)
    xb = jnp.maximum(dinv * (s_ref[1] + h_ref[1]) + b1_ref[:, _DH:], 0.0)
    x2 = jnp.concatenate([xa, xb], axis=1)
    h2 = jnp.dot(x2, w_ref[...], preferred_element_type=jnp.float32)
    ht = h2 * dinv
    out_ref[0] = ht[:, :_DH]
    out_ref[1] = ht[:, _DH:]


def _fin_body(s_ref, h_ref, dg_ref, b2_ref, out_ref):
    dinv = _dinv_from(dg_ref)
    oa = dinv * (s_ref[0] + h_ref[0]) + b2_ref[:, :_DH]
    ob = dinv * (s_ref[1] + h_ref[1]) + b2_ref[:, _DH:]
    out_ref[...] = jnp.concatenate([oa, ob], axis=1)


_half_spec = pl.BlockSpec((2, _B, _DH), lambda i: (0, i, 0))
_deg_spec = pl.BlockSpec((2, _B, _DW), lambda i: (0, i, 0))
_w_spec = pl.BlockSpec((_D, _D), lambda i: (0, 0))
_b_spec = pl.BlockSpec((1, _D), lambda i: (0, 0))
_half_out = jax.ShapeDtypeStruct((2, _N, _DH), jnp.float32)

_lin1_call = pl.pallas_call(
    _lin1_body,
    grid=(_G,),
    in_specs=[pl.BlockSpec((_B, _D), lambda i: (i, 0)), _w_spec, _deg_spec],
    out_specs=_half_spec,
    out_shape=_half_out,
)

_lin2_call = pl.pallas_call(
    _lin2_body,
    grid=(_G,),
    in_specs=[_half_spec, _half_spec, _deg_spec, _w_spec, _b_spec],
    out_specs=_half_spec,
    out_shape=_half_out,
)

_fin_call = pl.pallas_call(
    _fin_body,
    grid=(_G,),
    in_specs=[_half_spec, _half_spec, _deg_spec, _b_spec],
    out_specs=pl.BlockSpec((_B, _D), lambda i: (i, 0)),
    out_shape=jax.ShapeDtypeStruct((_N, _D), jnp.float32),
)


def kernel(x, edge_index, W1, b1, W2, b2):
    assert x.shape == (_N, _D) and edge_index.shape == (2, _E)
    src = edge_index[0].astype(jnp.int32)
    dst = edge_index[1].astype(jnp.int32)
    src2 = jnp.concatenate([src, src + _N])
    zeros_h = jnp.zeros((_RC, _DH), jnp.float32)
    zeros_d = jnp.zeros((_RC, _DW), jnp.float32)
    ones_d = jnp.ones((_KD, _DW), jnp.float32)

    deg2 = _deg_call(dst, ones_d, zeros_d).reshape(_NC, _NP, _DW)
    h1 = _lin1_call(x, W1, deg2)
    s1 = _agg_call(h1.reshape(_NC * _N, _DH), src2, dst, zeros_h)
    h2 = _lin2_call(s1.reshape(_NC, _NP, _DH), h1, deg2, W2,
                    b1.reshape(1, _D))
    s2 = _agg_call(h2.reshape(_NC * _N, _DH), src2, dst, zeros_h)
    return _fin_call(s2.reshape(_NC, _NP, _DH), h2, deg2, b2.reshape(1, _D))


# speedup vs baseline: 8.4460x; 1.1084x over previous
import jax
import jax.numpy as jnp
from jax import lax
from jax.experimental import pallas as pl
from jax.experimental.pallas import tpu as pltpu
from jax.experimental.pallas import tpu_sc as plsc

_N = 10000
_D = 256
_DH = 128
_E = 160000
_NC = 2
_NS = 16

_K = 128
_EPAD = 163840
_ET = _EPAD
_EPT = _EPAD // _NS
_NCH = _EPT // _K
_NCHD = _EPAD // (_NC * _NS) // _K
_NROW = _EPAD // _K

_NP = 10240
_RPT = _NP // _NS
_RC = 64
_NRC = _RPT // _RC

_MESH = plsc.VectorSubcoreMesh(core_axis_name="c", subcore_axis_name="s")


def _deg_body(dst2d_hbm, ones_hbm, zeros_hbm, out_hbm,
              acc, didx_v, ones_v, tmp_v, s0, s1, s2, s3):
    c = lax.axis_index("c")
    s = lax.axis_index("s")
    ss = (s0, s1, s2, s3)
    w = s * _NC + c
    pltpu.sync_copy(dst2d_hbm.at[pl.ds(w * _NCHD, _NCHD)], didx_v)
    pltpu.sync_copy(ones_hbm, ones_v)
    pltpu.sync_copy(zeros_hbm, tmp_v)
    for i in range(_NRC):
        pltpu.sync_copy(tmp_v, acc.at[pl.ds(s * _RPT + i * _RC, _RC)])
    plsc.subcore_barrier()

    def dfire(k, b):
        pltpu.async_copy(ones_v, acc.at[didx_v.at[k]], ss[b], add=True)

    def dwait(b):
        pltpu.make_async_copy(ones_hbm, ones_v, ss[b]).wait()

    for k in range(4):
        dfire(k, k)

    def main(j, carry):
        for t in range(4):
            dwait(t)
            dfire(4 * j + t, t)
        return carry

    lax.fori_loop(1, _NCHD // 4, main, 0)
    for b in range(4):
        dwait(b)
    plsc.subcore_barrier()
    for i in range(_NRC):
        r0 = s * _RPT + i * _RC
        pltpu.sync_copy(acc.at[pl.ds(r0, _RC)], tmp_v)
        pltpu.sync_copy(tmp_v, out_hbm.at[pl.ds(c * _NP + r0, _RC)])


_deg_call = pl.kernel(
    _deg_body,
    out_type=jax.ShapeDtypeStruct((_NC * _NP, _DH), jnp.float32),
    mesh=_MESH,
    scratch_types=[
        pltpu.VMEM_SHARED((_NP, _DH), jnp.float32),
        pltpu.VMEM((_NCHD, _K), jnp.int32),
        pltpu.VMEM((_K, _DH), jnp.float32),
        pltpu.VMEM((_RC, _DH), jnp.float32),
        pltpu.SemaphoreType.DMA,
        pltpu.SemaphoreType.DMA,
        pltpu.SemaphoreType.DMA,
        pltpu.SemaphoreType.DMA,
    ],
)


def _agg_body(h_hbm, idx2_hbm, zeros_hbm, out_hbm,
              acc, i0_v, i1_v, i2_v, i3_v, ra_v, rb_v, tmp_v,
              g0, g1, s0, s1, i0s, i1s, i2s, i3s):
    c = lax.axis_index("c")
    s = lax.axis_index("s")
    ibuf = (i0_v, i1_v, i2_v, i3_v)
    rows = (ra_v, rb_v)
    gs = (g0, g1)
    ss = (s0, s1)
    isem = (i0s, i1s, i2s, i3s)
    pltpu.sync_copy(zeros_hbm, tmp_v)
    for i in range(_NRC):
        pltpu.sync_copy(tmp_v, acc.at[pl.ds(s * _RPT + i * _RC, _RC)])
    plsc.subcore_barrier()
    rbase = s * _NCH

    def ifire(k, sl):
        pltpu.async_copy(idx2_hbm.at[c, rbase + k], ibuf[sl], isem[sl])

    def iwait(sl):
        pltpu.make_async_copy(idx2_hbm.at[0, 0], ibuf[sl], isem[sl]).wait()

    def gfire(sl, b):
        pltpu.async_copy(h_hbm.at[ibuf[sl].at[0]], rows[b], gs[b])

    def gwait(b):
        pltpu.make_async_copy(h_hbm.at[pl.ds(0, _K)], rows[b], gs[b]).wait()

    def sfire(sl, b):
        pltpu.async_copy(rows[b], acc.at[ibuf[sl].at[1]], ss[b], add=True)

    def swait(b):
        pltpu.make_async_copy(h_hbm.at[pl.ds(0, _K)], rows[b], ss[b]).wait()

    ifire(0, 0)
    ifire(1, 1)
    iwait(0)
    gfire(0, 0)
    ifire(2, 2)
    iwait(1)
    gfire(1, 1)
    gwait(0)
    sfire(0, 0)

    def main(j, carry):
        for t in range(4):
            k = 4 * j + 1 + t
            b = (1 + t) % 2
            swait(1 - b)
            ifire(k + 2, (3 + t) % 4)
            iwait((2 + t) % 4)
            gfire((2 + t) % 4, 1 - b)
            gwait(b)
            sfire((1 + t) % 4, b)
        return carry

    lax.fori_loop(0, (_NCH - 8) // 4, main, 0)
    for k in range(_NCH - 7, _NCH):
        b = k % 2
        swait(1 - b)
        if k + 2 < _NCH:
            ifire(k + 2, (k + 2) % 4)
        if k + 1 < _NCH:
            iwait((k + 1) % 4)
            gfire((k + 1) % 4, 1 - b)
        gwait(b)
        sfire(k % 4, b)
    swait((_NCH - 1) % 2)
    plsc.subcore_barrier()
    for i in range(_NRC):
        r0 = s * _RPT + i * _RC
        pltpu.sync_copy(acc.at[pl.ds(r0, _RC)], tmp_v)
        pltpu.sync_copy(tmp_v, out_hbm.at[pl.ds(c * _NP + r0, _RC)])


_agg_call = pl.kernel(
    _agg_body,
    out_type=jax.ShapeDtypeStruct((_NC * _NP, _DH), jnp.float32),
    mesh=_MESH,
    scratch_types=[
        pltpu.VMEM_SHARED((_NP, _DH), jnp.float32),
        pltpu.VMEM((2, _K), jnp.int32),
        pltpu.VMEM((2, _K), jnp.int32),
        pltpu.VMEM((2, _K), jnp.int32),
        pltpu.VMEM((2, _K), jnp.int32),
        pltpu.VMEM((_K, _DH), jnp.float32),
        pltpu.VMEM((_K, _DH), jnp.float32),
        pltpu.VMEM((_RC, _DH), jnp.float32),
        pltpu.SemaphoreType.DMA,
        pltpu.SemaphoreType.DMA,
        pltpu.SemaphoreType.DMA,
        pltpu.SemaphoreType.DMA,
        pltpu.SemaphoreType.DMA,
        pltpu.SemaphoreType.DMA,
        pltpu.SemaphoreType.DMA,
        pltpu.SemaphoreType.DMA,
    ],
)


_B = 1000
_G = _N // _B


def _dinv_from(dg_ref):
    dsum = dg_ref[0, :, 0:1] + dg_ref[1, :, 0:1] + 1.0
    return lax.rsqrt(dsum)


def _lin1_body(x_ref, w_ref, dg_ref, out_ref):
    h = jnp.dot(x_ref[...], w_ref[...], preferred_element_type=jnp.float32)
    ht = h * _dinv_from(dg_ref)
    out_ref[0] = ht[:, :_DH]
    out_ref[1] = ht[:, _DH:]


def _lin2_body(s_ref, h_ref, dg_ref, w_ref, b1_ref, out_ref):
    dinv = _dinv_from(dg_ref)
    xa = jnp.maximum(dinv * (s_ref[0] + h_ref[0]) + b1_ref[:, :_DH], 0.0)
    xb = jnp.maximum(dinv * (s_ref[1] + h_ref[1]) + b1_ref[:, _DH:], 0.0)
    x2 = jnp.concatenate([xa, xb], axis=1)
    h2 = jnp.dot(x2, w_ref[...], preferred_element_type=jnp.float32)
    ht = h2 * dinv
    out_ref[0] = ht[:, :_DH]
    out_ref[1] = ht[:, _DH:]


def _fin_body(s_ref, h_ref, dg_ref, b2_ref, out_ref):
    dinv = _dinv_from(dg_ref)
    oa = dinv * (s_ref[0] + h_ref[0]) + b2_ref[:, :_DH]
    ob = dinv * (s_ref[1] + h_ref[1]) + b2_ref[:, _DH:]
    out_ref[...] = jnp.concatenate([oa, ob], axis=1)


_half_spec = pl.BlockSpec((2, _B, _DH), lambda i: (0, i, 0))
_w_spec = pl.BlockSpec((_D, _D), lambda i: (0, 0))
_b_spec = pl.BlockSpec((1, _D), lambda i: (0, 0))
_half_out = jax.ShapeDtypeStruct((2, _N, _DH), jnp.float32)

_lin1_call = pl.pallas_call(
    _lin1_body,
    grid=(_G,),
    in_specs=[pl.BlockSpec((_B, _D), lambda i: (i, 0)), _w_spec, _half_spec],
    out_specs=_half_spec,
    out_shape=_half_out,
)

_lin2_call = pl.pallas_call(
    _lin2_body,
    grid=(_G,),
    in_specs=[_half_spec, _half_spec, _half_spec, _w_spec, _b_spec],
    out_specs=_half_spec,
    out_shape=_half_out,
)

_fin_call = pl.pallas_call(
    _fin_body,
    grid=(_G,),
    in_specs=[_half_spec, _half_spec, _half_spec, _b_spec],
    out_specs=pl.BlockSpec((_B, _D), lambda i: (i, 0)),
    out_shape=jax.ShapeDtypeStruct((_N, _D), jnp.float32),
)


def kernel(x, edge_index, W1, b1, W2, b2):
    assert x.shape == (_N, _D) and edge_index.shape == (2, _E)
    src = edge_index[0].astype(jnp.int32)
    dst = edge_index[1].astype(jnp.int32)
    pad = _EPAD - _E
    src_p = jnp.concatenate([src, jnp.zeros((pad,), jnp.int32)])
    dst_p = jnp.concatenate([dst, jnp.full((pad,), _N, jnp.int32)])
    srcv = src_p.reshape(_NROW, _K)
    dst2d = dst_p.reshape(_NROW, _K)
    idx2 = jnp.stack([jnp.stack([srcv, dst2d], axis=1),
                      jnp.stack([srcv + _N, dst2d], axis=1)])
    zeros_h = jnp.zeros((_RC, _DH), jnp.float32)
    ones_d = jnp.ones((_K, _DH), jnp.float32)

    deg2 = _deg_call(dst2d, ones_d, zeros_h).reshape(_NC, _NP, _DH)
    h1 = _lin1_call(x, W1, deg2)
    s1 = _agg_call(h1.reshape(_NC * _N, _DH), idx2, zeros_h)
    h2 = _lin2_call(s1.reshape(_NC, _NP, _DH), h1, deg2, W2,
                    b1.reshape(1, _D))
    s2 = _agg_call(h2.reshape(_NC * _N, _DH), idx2, zeros_h)
    return _fin_call(s2.reshape(_NC, _NP, _DH), h2, deg2, b2.reshape(1, _D))
